# baseline (device time: 147278 ns/iter reference)
import jax
import jax.numpy as jnp
from jax import lax
from jax.experimental import pallas as pl
from jax.experimental.pallas import tpu as pltpu

N_DEV = 32
B, Sq, Skv, Hq, Dh = 2, 256, 256, 128, 64
H_PER = Hq // N_DEV
HD = H_PER * Dh
ROWS = B * Sq
COLS = 512
CH = ROWS // N_DEV


def _ring_allreduce(p):

    def body(p_ref, out_ref, comm_ref, rs_send, rs_recv, ag_send, ag_recv):
        me = lax.axis_index("i")
        left = lax.rem(me - 1 + N_DEV, N_DEV)
        right = lax.rem(me + 1, N_DEV)

        barrier = pltpu.get_barrier_semaphore()
        for nbr in (left, right):
            pl.semaphore_signal(
                barrier, inc=1,
                device_id=(nbr,), device_id_type=pl.DeviceIdType.MESH,
            )
        pl.semaphore_wait(barrier, 2)

        out_ref[...] = p_ref[...]

        for h in range(N_DEV - 1):
            send_chunk = lax.rem(me - h + 2 * N_DEV, N_DEV)
            recv_chunk = lax.rem(me - h - 1 + 2 * N_DEV, N_DEV)
            rdma = pltpu.make_async_remote_copy(
                src_ref=out_ref.at[pl.ds(send_chunk * CH, CH)],
                dst_ref=comm_ref.at[h],
                send_sem=rs_send.at[h],
                recv_sem=rs_recv.at[h],
                device_id=(right,),
                device_id_type=pl.DeviceIdType.MESH,
            )
            rdma.start()
            rdma.wait()
            out_ref[pl.ds(recv_chunk * CH, CH)] = (
                out_ref[pl.ds(recv_chunk * CH, CH)] + comm_ref[h]
            )

        for g in range(N_DEV - 1):
            send_chunk = lax.rem(me + 1 - g + 2 * N_DEV, N_DEV)
            rdma = pltpu.make_async_remote_copy(
                src_ref=out_ref.at[pl.ds(send_chunk * CH, CH)],
                dst_ref=out_ref.at[pl.ds(send_chunk * CH, CH)],
                send_sem=ag_send.at[g],
                recv_sem=ag_recv.at[g],
                device_id=(right,),
                device_id_type=pl.DeviceIdType.MESH,
            )
            rdma.start()
            rdma.wait()

    return pl.pallas_call(
        body,
        out_shape=jax.ShapeDtypeStruct((ROWS, COLS), jnp.float32),
        in_specs=[pl.BlockSpec(memory_space=pltpu.VMEM)],
        out_specs=pl.BlockSpec(memory_space=pltpu.VMEM),
        scratch_shapes=[
            pltpu.VMEM((N_DEV - 1, CH, COLS), jnp.float32),
            pltpu.SemaphoreType.DMA((N_DEV - 1,)),
            pltpu.SemaphoreType.DMA((N_DEV - 1,)),
            pltpu.SemaphoreType.DMA((N_DEV - 1,)),
            pltpu.SemaphoreType.DMA((N_DEV - 1,)),
        ],
        compiler_params=pltpu.CompilerParams(collective_id=0),
    )(p)


def kernel(x, Wq, K_ext, V_ext, Wo):
    idx = lax.axis_index("i")

    Wq_l = lax.dynamic_slice_in_dim(Wq, idx * HD, HD, axis=1)
    Wo_l = lax.dynamic_slice_in_dim(Wo, idx * HD, HD, axis=0)

    xb = x.astype(jnp.bfloat16)
    Q = (xb @ Wq_l.astype(jnp.bfloat16)).reshape(B, Sq, H_PER, Dh)
    K = K_ext.astype(jnp.bfloat16)
    V = V_ext.astype(jnp.bfloat16)

    scores = jnp.einsum(
        "bihd,bjhd->bhij", Q, K, preferred_element_type=jnp.float32
    ) * 0.125
    qi = lax.broadcasted_iota(jnp.int32, (Sq, Skv), 0)
    ki = lax.broadcasted_iota(jnp.int32, (Sq, Skv), 1)
    mask = (jnp.abs(qi - ki) <= 128) | (ki < 32) | (qi < 32)
    scores = jnp.where(mask[None, None], scores, -1e9)
    w = jax.nn.softmax(scores, axis=-1)

    ctx = jnp.einsum(
        "bhij,bjhd->bihd", w.astype(jnp.bfloat16), V,
        preferred_element_type=jnp.float32,
    ).reshape(B, Sq, HD)

    partial = jnp.dot(
        ctx.astype(jnp.bfloat16), Wo_l.astype(jnp.bfloat16),
        preferred_element_type=jnp.float32,
    ).reshape(ROWS, COLS)

    return _ring_allreduce(partial).reshape(B, Sq, COLS)
